# Initial kernel scaffold; baseline (speedup 1.0000x reference)
import jax
import jax.numpy as jnp
from jax.experimental import pallas as pl


def kernel(emb_pos, emb_neg, W0, b0, W1, b1, attn_W, attn_b, q_W, sgn, u, i, j, edge_index):
    raise NotImplementedError("write your pallas kernel here")


# speedup vs baseline: 19.6779x; 19.6779x over previous
import functools

import jax
import jax.numpy as jnp
from jax import lax
from jax.experimental import pallas as pl
from jax.experimental.pallas import tpu as pltpu
from jax.experimental.pallas import tpu_sc as plsc

_NUM_U = 25000
_N = 50000
_DIM = 64
_E0 = 400000
_E = 800000
_B = 4096
_REG = 0.05
_GAMMA = 1e-10

_NC, _NS = 2, 16
_EPW = _E // (_NC * _NS)
_CH = 128
_NCHUNK = (_EPW + _CH - 1) // _CH
_EPAD = _NCHUNK * _CH
_HROWS = _NUM_U
_TRASH = _HROWS
_ACC_ROWS = _HROWS + 8
_ZR = 1568
_ZR_LAST = _HROWS - 15 * _ZR

_mesh = plsc.VectorSubcoreMesh(core_axis_name="c", subcore_axis_name="s")
_f32 = jnp.float32


@functools.partial(
    pl.kernel,
    compiler_params=pltpu.CompilerParams(use_tc_tiling_on_sc=False),
    out_type=jax.ShapeDtypeStruct((_N, 16), _f32),
    mesh=_mesh,
    scratch_types=[
        pltpu.VMEM((_NCHUNK, _CH), jnp.int32),
        pltpu.VMEM((_CH, 16), _f32),
        pltpu.VMEM_SHARED((_ACC_ROWS, 16), _f32),
        pltpu.SemaphoreType.DMA,
    ],
)
def _deg_sc(coll_pk, zeros16, ones16, deg_out, colv, onesv, dacc, sem):
    c = lax.axis_index("c")
    s = lax.axis_index("s")

    @pl.when(s < 15)
    def _():
        pltpu.sync_copy(zeros16, dacc.at[pl.ds(s * _ZR, _ZR)])

    @pl.when(s == 15)
    def _():
        pltpu.sync_copy(zeros16.at[pl.ds(0, _ZR_LAST)],
                        dacc.at[pl.ds(15 * _ZR, _ZR_LAST)])

    pltpu.sync_copy(coll_pk.at[c, s], colv)
    pltpu.sync_copy(ones16, onesv)
    plsc.subcore_barrier()

    @pl.loop(0, _NCHUNK, step=7)
    def _(k):
        for b in range(7):
            pltpu.async_copy(onesv, dacc.at[colv.at[k + b]], sem, add=True)
        for b in range(7):
            pltpu.make_async_copy(onesv, dacc.at[colv.at[k + b]], sem).wait()

    plsc.subcore_barrier()
    off = (1 - c) * _HROWS

    @pl.when(s < 15)
    def _():
        pltpu.sync_copy(dacc.at[pl.ds(s * _ZR, _ZR)],
                        deg_out.at[pl.ds(off + s * _ZR, _ZR)])

    @pl.when(s == 15)
    def _():
        pltpu.sync_copy(dacc.at[pl.ds(15 * _ZR, _ZR_LAST)],
                        deg_out.at[pl.ds(off + 15 * _ZR, _ZR_LAST)])


_HD = _DIM // 2


@functools.partial(
    pl.kernel,
    compiler_params=pltpu.CompilerParams(use_tc_tiling_on_sc=False),
    out_type=[jax.ShapeDtypeStruct((_N, _HD), _f32),
              jax.ShapeDtypeStruct((_N, _HD), _f32)],
    mesh=_mesh,
    scratch_types=[
        pltpu.VMEM((_NCHUNK, _CH), jnp.int32),
        pltpu.VMEM((_NCHUNK, _CH), jnp.int32),
        pltpu.VMEM((_CH, _HD), _f32),
        pltpu.VMEM((_CH, _HD), _f32),
        pltpu.VMEM_SHARED((_ACC_ROWS, _HD), _f32),
        pltpu.SemaphoreType.DMA,
        pltpu.SemaphoreType.DMA,
    ],
)
def _spmm_sc(ylo, yhi, colg_pk, rowl_pk, zeros32, tlo, thi,
             colv, rowv, vb0, vb1, acc, sem0, sem1):
    c = lax.axis_index("c")
    s = lax.axis_index("s")

    pltpu.sync_copy(colg_pk.at[c, s], colv)
    pltpu.sync_copy(rowl_pk.at[c, s], rowv)
    off = c * _HROWS

    for ytab, tout in ((ylo, tlo), (yhi, thi)):
        @pl.when(s < 15)
        def _():
            pltpu.sync_copy(zeros32, acc.at[pl.ds(s * _ZR, _ZR)])

        @pl.when(s == 15)
        def _():
            pltpu.sync_copy(zeros32.at[pl.ds(0, _ZR_LAST)],
                            acc.at[pl.ds(15 * _ZR, _ZR_LAST)])

        plsc.subcore_barrier()

        pltpu.async_copy(ytab.at[colv.at[0]], vb0, sem0)
        pltpu.async_copy(ytab.at[colv.at[1]], vb1, sem1)

        @pl.loop(0, _NCHUNK, step=2)
        def _(k):
            pltpu.make_async_copy(ytab.at[colv.at[k]], vb0, sem0).wait()
            pltpu.sync_copy(vb0, acc.at[rowv.at[k]], add=True)

            @pl.when(k + 2 < _NCHUNK)
            def _():
                pltpu.async_copy(ytab.at[colv.at[k + 2]], vb0, sem0)

            pltpu.make_async_copy(ytab.at[colv.at[k + 1]], vb1, sem1).wait()
            pltpu.sync_copy(vb1, acc.at[rowv.at[k + 1]], add=True)

            @pl.when(k + 3 < _NCHUNK)
            def _():
                pltpu.async_copy(ytab.at[colv.at[k + 3]], vb1, sem1)

        plsc.subcore_barrier()

        @pl.when(s < 15)
        def _():
            pltpu.sync_copy(acc.at[pl.ds(s * _ZR, _ZR)],
                            tout.at[pl.ds(off + s * _ZR, _ZR)])

        @pl.when(s == 15)
        def _():
            pltpu.sync_copy(acc.at[pl.ds(15 * _ZR, _ZR_LAST)],
                            tout.at[pl.ds(off + 15 * _ZR, _ZR_LAST)])

        plsc.subcore_barrier()


_G = 3 * _B
_GPW = _G // (_NC * _NS)
_GCH = _GPW // _CH


@functools.partial(
    pl.kernel,
    compiler_params=pltpu.CompilerParams(use_tc_tiling_on_sc=False),
    out_type=[
        jax.ShapeDtypeStruct((_G, _DIM), _f32),
        jax.ShapeDtypeStruct((_G, _DIM), _f32),
        jax.ShapeDtypeStruct((_G, _HD), _f32),
        jax.ShapeDtypeStruct((_G, _HD), _f32),
        jax.ShapeDtypeStruct((_G, _HD), _f32),
        jax.ShapeDtypeStruct((_G, _HD), _f32),
        jax.ShapeDtypeStruct((_G, 16), _f32),
    ],
    mesh=_mesh,
    scratch_types=[
        pltpu.VMEM((_GCH, _CH), jnp.int32),
        pltpu.VMEM((_CH, _DIM), _f32),
        pltpu.VMEM((_CH, _DIM), _f32),
        pltpu.VMEM((_CH, _HD), _f32),
        pltpu.VMEM((_CH, _HD), _f32),
        pltpu.VMEM((_CH, _HD), _f32),
        pltpu.VMEM((_CH, _HD), _f32),
        pltpu.VMEM((_CH, 16), _f32),
        pltpu.SemaphoreType.DMA,
    ],
)
def _gather_sc(idx_pk, pos_t, neg_t, t1lo_t, t1hi_t, t2lo_t, t2hi_t, dv_t,
               o_pos, o_neg, o_t1lo, o_t1hi, o_t2lo, o_t2hi, o_dv,
               idxv, bpos, bneg, b1lo, b1hi, b2lo, b2hi, bdv, sem):
    c = lax.axis_index("c")
    s = lax.axis_index("s")
    w = c * _NS + s
    pltpu.sync_copy(idx_pk.at[w], idxv)
    tabs = (pos_t, neg_t, t1lo_t, t1hi_t, t2lo_t, t2hi_t, dv_t)
    bufs = (bpos, bneg, b1lo, b1hi, b2lo, b2hi, bdv)
    outs = (o_pos, o_neg, o_t1lo, o_t1hi, o_t2lo, o_t2hi, o_dv)

    @pl.loop(0, _GCH)
    def _(k):
        for t, bf in zip(tabs, bufs):
            pltpu.async_copy(t.at[idxv.at[k]], bf, sem)
        for t, bf in zip(tabs, bufs):
            pltpu.make_async_copy(t.at[idxv.at[k]], bf, sem).wait()
        off = w * _GPW + k * _CH
        for bf, o in zip(bufs, outs):
            pltpu.sync_copy(bf, o.at[pl.ds(off, _CH)])


_RB = 2000


def _scale0_body(deg_ref, x_ref, dinv_ref, ylo_ref, yhi_ref):
    deg = deg_ref[...]
    dinv = jnp.where(deg > 0.0, lax.rsqrt(deg), 0.0)
    dinv_ref[...] = dinv
    d = dinv[:, 0:1]
    x = x_ref[...]
    ylo_ref[...] = d * x[:, :_HD]
    yhi_ref[...] = d * x[:, _HD:]


_scale0_tc = pl.pallas_call(
    _scale0_body,
    grid=(_N // _RB,),
    in_specs=[pl.BlockSpec((_RB, 16), lambda m: (m, 0)),
              pl.BlockSpec((_RB, _DIM), lambda m: (m, 0))],
    out_specs=[pl.BlockSpec((_RB, 16), lambda m: (m, 0)),
               pl.BlockSpec((_RB, _HD), lambda m: (m, 0)),
               pl.BlockSpec((_RB, _HD), lambda m: (m, 0))],
    out_shape=[jax.ShapeDtypeStruct((_N, 16), _f32),
               jax.ShapeDtypeStruct((_N, _HD), _f32),
               jax.ShapeDtypeStruct((_N, _HD), _f32)],
)


def _scale1_body(tlo_ref, thi_ref, dinv_ref, ylo_ref, yhi_ref):
    dv = dinv_ref[...][:, 0:1]
    d2 = dv * dv
    ylo_ref[...] = d2 * tlo_ref[...]
    yhi_ref[...] = d2 * thi_ref[...]


_scale1_tc = pl.pallas_call(
    _scale1_body,
    grid=(_N // _RB,),
    in_specs=[pl.BlockSpec((_RB, _HD), lambda m: (m, 0)),
              pl.BlockSpec((_RB, _HD), lambda m: (m, 0)),
              pl.BlockSpec((_RB, 16), lambda m: (m, 0))],
    out_specs=[pl.BlockSpec((_RB, _HD), lambda m: (m, 0)),
               pl.BlockSpec((_RB, _HD), lambda m: (m, 0))],
    out_shape=[jax.ShapeDtypeStruct((_N, _HD), _f32),
               jax.ShapeDtypeStruct((_N, _HD), _f32)],
)


_DB = 1024


def _dense_body(pu, alou, ahiu, blou, bhiu, du, nu,
                pi, aloi, ahii, bloi, bhii, di, ni,
                pj, aloj, ahij, bloj, bhij, dj, nj,
                sgn_ref, W0_ref, b0_ref, W1_ref, b1_ref,
                aW_ref, ab_ref, qv_ref, out_ref):
    W0 = W0_ref[...]
    b0 = b0_ref[...]
    W1 = W1_ref[...]
    b1 = b1_ref[...]
    aW = aW_ref[...]
    ab = ab_ref[...]
    qv = qv_ref[...]

    def seg(p_ref, alo_ref, ahi_ref, blo_ref, bhi_ref, d_ref, n_ref):
        d = d_ref[...][:, 0:1]
        t1 = jnp.concatenate([alo_ref[...], ahi_ref[...]], axis=1)
        t2 = jnp.concatenate([blo_ref[...], bhi_ref[...]], axis=1)
        zp = (p_ref[...] + d * t1 + d * t2) * (1.0 / 3.0)
        h = jnp.maximum(
            jnp.dot(n_ref[...], W0, preferred_element_type=_f32) + b0, 0.0)
        zn = jnp.maximum(
            jnp.dot(h, W1, preferred_element_type=_f32) + b1, 0.0)
        wp = jnp.sum(jnp.tanh(
            jnp.dot(zp, aW, preferred_element_type=_f32) + ab) * qv,
            axis=1, keepdims=True)
        wn = jnp.sum(jnp.tanh(
            jnp.dot(zn, aW, preferred_element_type=_f32) + ab) * qv,
            axis=1, keepdims=True)
        m = jnp.maximum(wp, wn)
        ep = jnp.exp(wp - m)
        en = jnp.exp(wn - m)
        al = ep / (ep + en)
        return al * zp + (1.0 - al) * zn

    Zu = seg(pu, alou, ahiu, blou, bhiu, du, nu)
    Zi = seg(pi, aloi, ahii, bloi, bhii, di, ni)
    Zj = seg(pj, aloj, ahij, bloj, bhij, dj, nj)
    ps = jnp.sum(Zu * Zi, axis=1, keepdims=True)
    ns = jnp.sum(Zu * Zj, axis=1, keepdims=True)
    x = sgn_ref[...] * ps - ns
    sig = 1.0 / (1.0 + jnp.exp(-x))
    part_sbpr = jnp.sum(jnp.log(_GAMMA + sig), axis=(0, 1), keepdims=True)
    part_reg = jnp.sum(Zu * Zu + Zi * Zi + Zj * Zj, axis=(0, 1), keepdims=True)
    part = (-part_sbpr + _REG * part_reg) / _B

    @pl.when(pl.program_id(0) == 0)
    def _():
        out_ref[...] = jnp.zeros((1, 1), _f32)

    out_ref[...] += part


_row_spec = pl.BlockSpec((_DB, _DIM), lambda m: (m, 0))
_half_spec = pl.BlockSpec((_DB, _HD), lambda m: (m, 0))
_dv_spec = pl.BlockSpec((_DB, 16), lambda m: (m, 0))
_w_spec = pl.BlockSpec((_DIM, _DIM), lambda m: (0, 0))
_b_spec = pl.BlockSpec((1, _DIM), lambda m: (0, 0))

_dense_tc = pl.pallas_call(
    _dense_body,
    grid=(_B // _DB,),
    in_specs=[_row_spec, _half_spec, _half_spec, _half_spec, _half_spec,
              _dv_spec, _row_spec] * 3
    + [pl.BlockSpec((_DB, 1), lambda m: (m, 0)),
       _w_spec, _b_spec, _w_spec, _b_spec, _w_spec, _b_spec, _b_spec],
    out_specs=pl.BlockSpec((1, 1), lambda m: (0, 0)),
    out_shape=jax.ShapeDtypeStruct((1, 1), _f32),
)


def _pack_edges(a, pad):
    a = a.reshape(_NC, _NS, _EPW)
    a = jnp.pad(a, ((0, 0), (0, 0), (0, _EPAD - _EPW)), constant_values=pad)
    return a.reshape(_NC, _NS, _NCHUNK, _CH)


def kernel(emb_pos, emb_neg, W0, b0, W1, b1, attn_W, attn_b, q_W,
           sgn, u, i, j, edge_index):
    row, col = edge_index[0], edge_index[1]
    row_loc = jnp.concatenate([row[:_E0], row[_E0:] - _NUM_U])
    col_loc = jnp.concatenate([col[:_E0] - _NUM_U, col[_E0:]])
    colg_pk = _pack_edges(col, 0)
    rowl_pk = _pack_edges(row_loc, _TRASH)
    coll_pk = _pack_edges(col_loc, _TRASH)

    zeros32 = jnp.zeros((_ZR, _HD), _f32)
    zeros16 = jnp.zeros((_ZR, 16), _f32)
    ones16 = jnp.ones((_CH, 16), _f32)

    deg16 = _deg_sc(coll_pk, zeros16, ones16)
    dinv16, y0lo, y0hi = _scale0_tc(deg16, emb_pos)
    t1lo, t1hi = _spmm_sc(y0lo, y0hi, colg_pk, rowl_pk, zeros32)
    y1lo, y1hi = _scale1_tc(t1lo, t1hi, dinv16)
    t2lo, t2hi = _spmm_sc(y1lo, y1hi, colg_pk, rowl_pk, zeros32)

    idx_pk = jnp.concatenate([u, _NUM_U + i, _NUM_U + j]).reshape(
        _NC * _NS, _GCH, _CH)
    posg, negg, t1log, t1hig, t2log, t2hig, dvg = _gather_sc(
        idx_pk, emb_pos, emb_neg, t1lo, t1hi, t2lo, t2hi, dinv16)

    sgn2 = sgn.reshape(_B, 1)
    segs = []
    for a in range(3):
        sl = slice(a * _B, (a + 1) * _B)
        segs += [posg[sl], t1log[sl], t1hig[sl], t2log[sl], t2hig[sl],
                 dvg[sl], negg[sl]]
    out = _dense_tc(
        *segs,
        sgn2, W0, b0.reshape(1, _DIM), W1, b1.reshape(1, _DIM),
        attn_W, attn_b.reshape(1, _DIM), q_W.reshape(1, _DIM))
    return out[0, 0]


# speedup vs baseline: 20.9747x; 1.0659x over previous
import functools

import jax
import jax.numpy as jnp
from jax import lax
from jax.experimental import pallas as pl
from jax.experimental.pallas import tpu as pltpu
from jax.experimental.pallas import tpu_sc as plsc

_NUM_U = 25000
_N = 50000
_DIM = 64
_E0 = 400000
_E = 800000
_B = 4096
_REG = 0.05
_GAMMA = 1e-10

_NC, _NS = 2, 16
_EPW = _E // (_NC * _NS)
_CH = 128
_NCHUNK = (_EPW + _CH - 1) // _CH
_EPAD = _NCHUNK * _CH
_HROWS = _NUM_U
_TRASH = _HROWS
_ACC_ROWS = _HROWS + 8
_ZR = 1568
_ZR_LAST = _HROWS - 15 * _ZR
_HD = _DIM // 2

_mesh = plsc.VectorSubcoreMesh(core_axis_name="c", subcore_axis_name="s")
_f32 = jnp.float32


@functools.partial(
    pl.kernel,
    compiler_params=pltpu.CompilerParams(use_tc_tiling_on_sc=False),
    out_type=jax.ShapeDtypeStruct((_N, 16), _f32),
    mesh=_mesh,
    scratch_types=[
        pltpu.VMEM((_NCHUNK, _CH), jnp.int32),
        pltpu.VMEM((_CH, 16), _f32),
        pltpu.VMEM_SHARED((_ACC_ROWS, 16), _f32),
        pltpu.SemaphoreType.DMA,
    ],
)
def _deg_sc(coll_pk, zeros16, ones16, deg_out, colv, onesv, dacc, sem):
    c = lax.axis_index("c")
    s = lax.axis_index("s")

    @pl.when(s < 15)
    def _():
        pltpu.sync_copy(zeros16, dacc.at[pl.ds(s * _ZR, _ZR)])

    @pl.when(s == 15)
    def _():
        pltpu.sync_copy(zeros16.at[pl.ds(0, _ZR_LAST)],
                        dacc.at[pl.ds(15 * _ZR, _ZR_LAST)])

    pltpu.sync_copy(coll_pk.at[c, s], colv)
    pltpu.sync_copy(ones16, onesv)
    plsc.subcore_barrier()

    @pl.loop(0, _NCHUNK, step=7)
    def _(k):
        for b in range(7):
            pltpu.async_copy(onesv, dacc.at[colv.at[k + b]], sem, add=True)
        for b in range(7):
            pltpu.make_async_copy(onesv, dacc.at[colv.at[k + b]], sem).wait()

    plsc.subcore_barrier()
    off = (1 - c) * _HROWS

    @pl.when(s < 15)
    def _():
        pltpu.sync_copy(dacc.at[pl.ds(s * _ZR, _ZR)],
                        deg_out.at[pl.ds(off + s * _ZR, _ZR)])

    @pl.when(s == 15)
    def _():
        pltpu.sync_copy(dacc.at[pl.ds(15 * _ZR, _ZR_LAST)],
                        deg_out.at[pl.ds(off + 15 * _ZR, _ZR_LAST)])


@functools.partial(
    pl.kernel,
    compiler_params=pltpu.CompilerParams(use_tc_tiling_on_sc=False),
    out_type=[jax.ShapeDtypeStruct((_N, _HD), _f32),
              jax.ShapeDtypeStruct((_N, _HD), _f32)],
    mesh=_mesh,
    scratch_types=[
        pltpu.VMEM((_NCHUNK, _CH), jnp.int32),
        pltpu.VMEM((_NCHUNK, _CH), jnp.int32),
        pltpu.VMEM((_CH, _HD), _f32),
        pltpu.VMEM((_CH, _HD), _f32),
        pltpu.VMEM_SHARED((_ACC_ROWS, _HD), _f32),
        pltpu.SemaphoreType.DMA,
        pltpu.SemaphoreType.DMA,
    ],
)
def _spmm_sc(ylo, yhi, colg_pk, rowl_pk, zeros32, tlo, thi,
             colv, rowv, vb0, vb1, acc, sem0, sem1):
    c = lax.axis_index("c")
    s = lax.axis_index("s")

    pltpu.sync_copy(colg_pk.at[c, s], colv)
    pltpu.sync_copy(rowl_pk.at[c, s], rowv)
    off = c * _HROWS

    for ytab, tout in ((ylo, tlo), (yhi, thi)):
        @pl.when(s < 15)
        def _():
            pltpu.sync_copy(zeros32, acc.at[pl.ds(s * _ZR, _ZR)])

        @pl.when(s == 15)
        def _():
            pltpu.sync_copy(zeros32.at[pl.ds(0, _ZR_LAST)],
                            acc.at[pl.ds(15 * _ZR, _ZR_LAST)])

        plsc.subcore_barrier()

        pltpu.async_copy(ytab.at[colv.at[0]], vb0, sem0)
        pltpu.async_copy(ytab.at[colv.at[1]], vb1, sem1)

        @pl.loop(0, _NCHUNK, step=2)
        def _(k):
            pltpu.make_async_copy(ytab.at[colv.at[k]], vb0, sem0).wait()
            pltpu.sync_copy(vb0, acc.at[rowv.at[k]], add=True)

            @pl.when(k + 2 < _NCHUNK)
            def _():
                pltpu.async_copy(ytab.at[colv.at[k + 2]], vb0, sem0)

            pltpu.make_async_copy(ytab.at[colv.at[k + 1]], vb1, sem1).wait()
            pltpu.sync_copy(vb1, acc.at[rowv.at[k + 1]], add=True)

            @pl.when(k + 3 < _NCHUNK)
            def _():
                pltpu.async_copy(ytab.at[colv.at[k + 3]], vb1, sem1)

        plsc.subcore_barrier()

        @pl.when(s < 15)
        def _():
            pltpu.sync_copy(acc.at[pl.ds(s * _ZR, _ZR)],
                            tout.at[pl.ds(off + s * _ZR, _ZR)])

        @pl.when(s == 15)
        def _():
            pltpu.sync_copy(acc.at[pl.ds(15 * _ZR, _ZR_LAST)],
                            tout.at[pl.ds(off + 15 * _ZR, _ZR_LAST)])

        plsc.subcore_barrier()


_G = 3 * _B
_GPW = _G // (_NC * _NS)
_GCH = _GPW // _CH


@functools.partial(
    pl.kernel,
    compiler_params=pltpu.CompilerParams(use_tc_tiling_on_sc=False),
    out_type=[
        jax.ShapeDtypeStruct((_G, _DIM), _f32),
        jax.ShapeDtypeStruct((_G, _DIM), _f32),
        jax.ShapeDtypeStruct((_G, _HD), _f32),
        jax.ShapeDtypeStruct((_G, _HD), _f32),
        jax.ShapeDtypeStruct((_G, _HD), _f32),
        jax.ShapeDtypeStruct((_G, _HD), _f32),
        jax.ShapeDtypeStruct((_G, 16), _f32),
    ],
    mesh=_mesh,
    scratch_types=[
        pltpu.VMEM((_GCH, _CH), jnp.int32),
        pltpu.VMEM((_CH, _DIM), _f32),
        pltpu.VMEM((_CH, _DIM), _f32),
        pltpu.VMEM((_CH, _HD), _f32),
        pltpu.VMEM((_CH, _HD), _f32),
        pltpu.VMEM((_CH, _HD), _f32),
        pltpu.VMEM((_CH, _HD), _f32),
        pltpu.VMEM((_CH, 16), _f32),
        pltpu.SemaphoreType.DMA,
    ],
)
def _gather_sc(idx_pk, pos_t, neg_t, t1lo_t, t1hi_t, t2lo_t, t2hi_t, dv_t,
               o_pos, o_neg, o_t1lo, o_t1hi, o_t2lo, o_t2hi, o_dv,
               idxv, bpos, bneg, b1lo, b1hi, b2lo, b2hi, bdv, sem):
    c = lax.axis_index("c")
    s = lax.axis_index("s")
    w = c * _NS + s
    pltpu.sync_copy(idx_pk.at[w], idxv)
    tabs = (pos_t, neg_t, t1lo_t, t1hi_t, t2lo_t, t2hi_t, dv_t)
    bufs = (bpos, bneg, b1lo, b1hi, b2lo, b2hi, bdv)
    outs = (o_pos, o_neg, o_t1lo, o_t1hi, o_t2lo, o_t2hi, o_dv)

    @pl.loop(0, _GCH)
    def _(k):
        for t, bf in zip(tabs, bufs):
            pltpu.async_copy(t.at[idxv.at[k]], bf, sem)
        for t, bf in zip(tabs, bufs):
            pltpu.make_async_copy(t.at[idxv.at[k]], bf, sem).wait()
        off = w * _GPW + k * _CH
        for bf, o in zip(bufs, outs):
            pltpu.sync_copy(bf, o.at[pl.ds(off, _CH)])


_RB = 2000


def _scale0_body(deg_ref, x_ref, dinv_ref, ylo_ref, yhi_ref):
    deg = deg_ref[...]
    dinv = jnp.where(deg > 0.0, lax.rsqrt(deg), 0.0)
    dinv_ref[...] = dinv
    d = dinv[:, 0:1]
    x = x_ref[...]
    ylo_ref[...] = d * x[:, :_HD]
    yhi_ref[...] = d * x[:, _HD:]


_scale0_tc = pl.pallas_call(
    _scale0_body,
    grid=(_N // _RB,),
    in_specs=[pl.BlockSpec((_RB, 16), lambda m: (m, 0)),
              pl.BlockSpec((_RB, _DIM), lambda m: (m, 0))],
    out_specs=[pl.BlockSpec((_RB, 16), lambda m: (m, 0)),
               pl.BlockSpec((_RB, _HD), lambda m: (m, 0)),
               pl.BlockSpec((_RB, _HD), lambda m: (m, 0))],
    out_shape=[jax.ShapeDtypeStruct((_N, 16), _f32),
               jax.ShapeDtypeStruct((_N, _HD), _f32),
               jax.ShapeDtypeStruct((_N, _HD), _f32)],
)


def _scale1_body(tlo_ref, thi_ref, dinv_ref, ylo_ref, yhi_ref):
    dv = dinv_ref[...][:, 0:1]
    d2 = dv * dv
    ylo_ref[...] = d2 * tlo_ref[...]
    yhi_ref[...] = d2 * thi_ref[...]


_scale1_tc = pl.pallas_call(
    _scale1_body,
    grid=(_N // _RB,),
    in_specs=[pl.BlockSpec((_RB, _HD), lambda m: (m, 0)),
              pl.BlockSpec((_RB, _HD), lambda m: (m, 0)),
              pl.BlockSpec((_RB, 16), lambda m: (m, 0))],
    out_specs=[pl.BlockSpec((_RB, _HD), lambda m: (m, 0)),
               pl.BlockSpec((_RB, _HD), lambda m: (m, 0))],
    out_shape=[jax.ShapeDtypeStruct((_N, _HD), _f32),
               jax.ShapeDtypeStruct((_N, _HD), _f32)],
)


_DB = 1024


def _dense_body(pu, alou, ahiu, blou, bhiu, du, nu,
                pi, aloi, ahii, bloi, bhii, di, ni,
                pj, aloj, ahij, bloj, bhij, dj, nj,
                sgn_ref, W0_ref, b0_ref, W1_ref, b1_ref,
                aW_ref, ab_ref, qv_ref, out_ref):
    W0 = W0_ref[...]
    b0 = b0_ref[...]
    W1 = W1_ref[...]
    b1 = b1_ref[...]
    aW = aW_ref[...]
    ab = ab_ref[...]
    qv = qv_ref[...]

    def seg(p_ref, alo_ref, ahi_ref, blo_ref, bhi_ref, d_ref, n_ref):
        d = d_ref[...][:, 0:1]
        t1 = jnp.concatenate([alo_ref[...], ahi_ref[...]], axis=1)
        t2 = jnp.concatenate([blo_ref[...], bhi_ref[...]], axis=1)
        zp = (p_ref[...] + d * t1 + d * t2) * (1.0 / 3.0)
        h = jnp.maximum(
            jnp.dot(n_ref[...], W0, preferred_element_type=_f32) + b0, 0.0)
        zn = jnp.maximum(
            jnp.dot(h, W1, preferred_element_type=_f32) + b1, 0.0)
        wp = jnp.sum(jnp.tanh(
            jnp.dot(zp, aW, preferred_element_type=_f32) + ab) * qv,
            axis=1, keepdims=True)
        wn = jnp.sum(jnp.tanh(
            jnp.dot(zn, aW, preferred_element_type=_f32) + ab) * qv,
            axis=1, keepdims=True)
        m = jnp.maximum(wp, wn)
        ep = jnp.exp(wp - m)
        en = jnp.exp(wn - m)
        al = ep / (ep + en)
        return al * zp + (1.0 - al) * zn

    Zu = seg(pu, alou, ahiu, blou, bhiu, du, nu)
    Zi = seg(pi, aloi, ahii, bloi, bhii, di, ni)
    Zj = seg(pj, aloj, ahij, bloj, bhij, dj, nj)
    ps = jnp.sum(Zu * Zi, axis=1, keepdims=True)
    ns = jnp.sum(Zu * Zj, axis=1, keepdims=True)
    x = sgn_ref[...] * ps - ns
    sig = 1.0 / (1.0 + jnp.exp(-x))
    part_sbpr = jnp.sum(jnp.log(_GAMMA + sig), axis=(0, 1), keepdims=True)
    part_reg = jnp.sum(Zu * Zu + Zi * Zi + Zj * Zj, axis=(0, 1), keepdims=True)
    part = (-part_sbpr + _REG * part_reg) / _B

    @pl.when(pl.program_id(0) == 0)
    def _():
        out_ref[...] = jnp.zeros((1, 1), _f32)

    out_ref[...] += part


def _seg_spec(cols, a):
    return pl.BlockSpec((_DB, cols), lambda m, a=a: (m + a * (_B // _DB), 0))


_w_spec = pl.BlockSpec((_DIM, _DIM), lambda m: (0, 0))
_b_spec = pl.BlockSpec((1, _DIM), lambda m: (0, 0))

_dense_tc = pl.pallas_call(
    _dense_body,
    grid=(_B // _DB,),
    in_specs=[sp
              for a in range(3)
              for sp in (_seg_spec(_DIM, a), _seg_spec(_HD, a),
                         _seg_spec(_HD, a), _seg_spec(_HD, a),
                         _seg_spec(_HD, a), _seg_spec(16, a),
                         _seg_spec(_DIM, a))]
    + [pl.BlockSpec((_DB, 1), lambda m: (m, 0)),
       _w_spec, _b_spec, _w_spec, _b_spec, _w_spec, _b_spec, _b_spec],
    out_specs=pl.BlockSpec((1, 1), lambda m: (0, 0)),
    out_shape=jax.ShapeDtypeStruct((1, 1), _f32),
)


def _pack_edges(a, pad):
    a = a.reshape(_NC, _NS, _EPW)
    a = jnp.pad(a, ((0, 0), (0, 0), (0, _EPAD - _EPW)), constant_values=pad)
    return a.reshape(_NC, _NS, _NCHUNK, _CH)


def kernel(emb_pos, emb_neg, W0, b0, W1, b1, attn_W, attn_b, q_W,
           sgn, u, i, j, edge_index):
    row, col = edge_index[0], edge_index[1]
    row_loc = lax.rem(row, _NUM_U)
    col_loc = lax.rem(col, _NUM_U)
    colg_pk = _pack_edges(col, 0)
    rowl_pk = _pack_edges(row_loc, _TRASH)
    coll_pk = _pack_edges(col_loc, _TRASH)

    zeros32 = jnp.zeros((_ZR, _HD), _f32)
    zeros16 = jnp.zeros((_ZR, 16), _f32)
    ones16 = jnp.ones((_CH, 16), _f32)

    deg16 = _deg_sc(coll_pk, zeros16, ones16)
    dinv16, y0lo, y0hi = _scale0_tc(deg16, emb_pos)
    t1lo, t1hi = _spmm_sc(y0lo, y0hi, colg_pk, rowl_pk, zeros32)
    y1lo, y1hi = _scale1_tc(t1lo, t1hi, dinv16)
    t2lo, t2hi = _spmm_sc(y1lo, y1hi, colg_pk, rowl_pk, zeros32)

    idx_pk = jnp.concatenate([u, _NUM_U + i, _NUM_U + j]).reshape(
        _NC * _NS, _GCH, _CH)
    posg, negg, t1log, t1hig, t2log, t2hig, dvg = _gather_sc(
        idx_pk, emb_pos, emb_neg, t1lo, t1hi, t2lo, t2hi, dinv16)

    sgn2 = sgn.reshape(_B, 1)
    segs = [posg, t1log, t1hig, t2log, t2hig, dvg, negg] * 3
    out = _dense_tc(
        *segs,
        sgn2, W0, b0.reshape(1, _DIM), W1, b1.reshape(1, _DIM),
        attn_W, attn_b.reshape(1, _DIM), q_W.reshape(1, _DIM))
    return out[0, 0]


# speedup vs baseline: 24.2360x; 1.1555x over previous
import functools

import jax
import jax.numpy as jnp
from jax import lax
from jax.experimental import pallas as pl
from jax.experimental.pallas import tpu as pltpu
from jax.experimental.pallas import tpu_sc as plsc

_NUM_U = 25000
_N = 50000
_DIM = 64
_E0 = 400000
_E = 800000
_B = 4096
_REG = 0.05
_GAMMA = 1e-10

_NC, _NS = 2, 16
_EPW = _E // (_NC * _NS)
_CH = 128
_NCHUNK = (_EPW + _CH - 1) // _CH
_EPAD = _NCHUNK * _CH
_HROWS = _NUM_U
_TRASH = _HROWS
_ACC_ROWS = _HROWS + 8
_ZR = 1568
_ZR_LAST = _HROWS - 15 * _ZR
_HD = _DIM // 2

_mesh = plsc.VectorSubcoreMesh(core_axis_name="c", subcore_axis_name="s")
_f32 = jnp.float32


@functools.partial(
    pl.kernel,
    compiler_params=pltpu.CompilerParams(use_tc_tiling_on_sc=False),
    out_type=jax.ShapeDtypeStruct((_N, 16), _f32),
    mesh=_mesh,
    scratch_types=[
        pltpu.VMEM((_NCHUNK, _CH), jnp.int32),
        pltpu.VMEM((_CH, 16), _f32),
        pltpu.VMEM_SHARED((_ACC_ROWS, 16), _f32),
        pltpu.SemaphoreType.DMA,
    ],
)
def _deg_sc(coll_pk, zeros16, ones16, deg_out, colv, onesv, dacc, sem):
    c = lax.axis_index("c")
    s = lax.axis_index("s")

    @pl.when(s < 15)
    def _():
        pltpu.sync_copy(zeros16, dacc.at[pl.ds(s * _ZR, _ZR)])

    @pl.when(s == 15)
    def _():
        pltpu.sync_copy(zeros16.at[pl.ds(0, _ZR_LAST)],
                        dacc.at[pl.ds(15 * _ZR, _ZR_LAST)])

    pltpu.sync_copy(coll_pk.at[c, s], colv)
    pltpu.sync_copy(ones16, onesv)
    plsc.subcore_barrier()

    @pl.loop(0, _NCHUNK, step=7)
    def _(k):
        for b in range(7):
            pltpu.async_copy(onesv, dacc.at[colv.at[k + b]], sem, add=True)
        for b in range(7):
            pltpu.make_async_copy(onesv, dacc.at[colv.at[k + b]], sem).wait()

    plsc.subcore_barrier()
    off = (1 - c) * _HROWS

    @pl.when(s < 15)
    def _():
        pltpu.sync_copy(dacc.at[pl.ds(s * _ZR, _ZR)],
                        deg_out.at[pl.ds(off + s * _ZR, _ZR)])

    @pl.when(s == 15)
    def _():
        pltpu.sync_copy(dacc.at[pl.ds(15 * _ZR, _ZR_LAST)],
                        deg_out.at[pl.ds(off + 15 * _ZR, _ZR_LAST)])


@functools.partial(
    pl.kernel,
    compiler_params=pltpu.CompilerParams(use_tc_tiling_on_sc=False),
    out_type=[jax.ShapeDtypeStruct((_N, _HD), _f32),
              jax.ShapeDtypeStruct((_N, _HD), _f32)],
    mesh=_mesh,
    scratch_types=[
        pltpu.VMEM((_NCHUNK, _CH), jnp.int32),
        pltpu.VMEM((_NCHUNK, _CH), jnp.int32),
        [pltpu.VMEM((_CH, _HD), _f32)] * 4,
        pltpu.VMEM_SHARED((_ACC_ROWS, _HD), _f32),
        [pltpu.SemaphoreType.DMA] * 4,
        [pltpu.SemaphoreType.DMA] * 4,
    ],
)
def _spmm_sc(ylo, yhi, colg_pk, rowl_pk, zeros32, tlo, thi,
             colv, rowv, vbs, acc, gsems, ssems):
    c = lax.axis_index("c")
    s = lax.axis_index("s")

    pltpu.sync_copy(colg_pk.at[c, s], colv)
    pltpu.sync_copy(rowl_pk.at[c, s], rowv)
    off = c * _HROWS

    for ytab, tout in ((ylo, tlo), (yhi, thi)):
        @pl.when(s < 15)
        def _():
            pltpu.sync_copy(zeros32, acc.at[pl.ds(s * _ZR, _ZR)])

        @pl.when(s == 15)
        def _():
            pltpu.sync_copy(zeros32.at[pl.ds(0, _ZR_LAST)],
                            acc.at[pl.ds(15 * _ZR, _ZR_LAST)])

        plsc.subcore_barrier()

        for b in range(4):
            pltpu.async_copy(ytab.at[colv.at[b]], vbs[b], gsems[b])

        @pl.loop(0, _NCHUNK, step=4)
        def _(k):
            for b in range(4):
                pltpu.make_async_copy(
                    ytab.at[colv.at[k + b]], vbs[b], gsems[b]).wait()
                pltpu.async_copy(
                    vbs[b], acc.at[rowv.at[k + b]], ssems[b], add=True)
            for b in range(4):
                pltpu.make_async_copy(
                    vbs[b], acc.at[rowv.at[k + b]], ssems[b]).wait()

                @pl.when(k + b + 4 < _NCHUNK)
                def _(b=b):
                    pltpu.async_copy(
                        ytab.at[colv.at[k + b + 4]], vbs[b], gsems[b])

        plsc.subcore_barrier()

        @pl.when(s < 15)
        def _():
            pltpu.sync_copy(acc.at[pl.ds(s * _ZR, _ZR)],
                            tout.at[pl.ds(off + s * _ZR, _ZR)])

        @pl.when(s == 15)
        def _():
            pltpu.sync_copy(acc.at[pl.ds(15 * _ZR, _ZR_LAST)],
                            tout.at[pl.ds(off + 15 * _ZR, _ZR_LAST)])

        plsc.subcore_barrier()


_G = 3 * _B
_GPW = _G // (_NC * _NS)
_GCH = _GPW // _CH


@functools.partial(
    pl.kernel,
    compiler_params=pltpu.CompilerParams(use_tc_tiling_on_sc=False),
    out_type=[
        jax.ShapeDtypeStruct((_G, _DIM), _f32),
        jax.ShapeDtypeStruct((_G, _DIM), _f32),
        jax.ShapeDtypeStruct((_G, _HD), _f32),
        jax.ShapeDtypeStruct((_G, _HD), _f32),
        jax.ShapeDtypeStruct((_G, _HD), _f32),
        jax.ShapeDtypeStruct((_G, _HD), _f32),
        jax.ShapeDtypeStruct((_G, 16), _f32),
    ],
    mesh=_mesh,
    scratch_types=[
        pltpu.VMEM((_GCH, _CH), jnp.int32),
        pltpu.VMEM((_CH, _DIM), _f32),
        pltpu.VMEM((_CH, _DIM), _f32),
        pltpu.VMEM((_CH, _HD), _f32),
        pltpu.VMEM((_CH, _HD), _f32),
        pltpu.VMEM((_CH, _HD), _f32),
        pltpu.VMEM((_CH, _HD), _f32),
        pltpu.VMEM((_CH, 16), _f32),
        pltpu.SemaphoreType.DMA,
    ],
)
def _gather_sc(idx_pk, pos_t, neg_t, t1lo_t, t1hi_t, t2lo_t, t2hi_t, dv_t,
               o_pos, o_neg, o_t1lo, o_t1hi, o_t2lo, o_t2hi, o_dv,
               idxv, bpos, bneg, b1lo, b1hi, b2lo, b2hi, bdv, sem):
    c = lax.axis_index("c")
    s = lax.axis_index("s")
    w = c * _NS + s
    pltpu.sync_copy(idx_pk.at[w], idxv)
    tabs = (pos_t, neg_t, t1lo_t, t1hi_t, t2lo_t, t2hi_t, dv_t)
    bufs = (bpos, bneg, b1lo, b1hi, b2lo, b2hi, bdv)
    outs = (o_pos, o_neg, o_t1lo, o_t1hi, o_t2lo, o_t2hi, o_dv)

    @pl.loop(0, _GCH)
    def _(k):
        for t, bf in zip(tabs, bufs):
            pltpu.async_copy(t.at[idxv.at[k]], bf, sem)
        for t, bf in zip(tabs, bufs):
            pltpu.make_async_copy(t.at[idxv.at[k]], bf, sem).wait()
        off = w * _GPW + k * _CH
        for bf, o in zip(bufs, outs):
            pltpu.sync_copy(bf, o.at[pl.ds(off, _CH)])


_RB = 2000


def _scale0_body(deg_ref, x_ref, dinv_ref, ylo_ref, yhi_ref):
    deg = deg_ref[...]
    dinv = jnp.where(deg > 0.0, lax.rsqrt(deg), 0.0)
    dinv_ref[...] = dinv
    d = dinv[:, 0:1]
    x = x_ref[...]
    ylo_ref[...] = d * x[:, :_HD]
    yhi_ref[...] = d * x[:, _HD:]


_scale0_tc = pl.pallas_call(
    _scale0_body,
    grid=(_N // _RB,),
    in_specs=[pl.BlockSpec((_RB, 16), lambda m: (m, 0)),
              pl.BlockSpec((_RB, _DIM), lambda m: (m, 0))],
    out_specs=[pl.BlockSpec((_RB, 16), lambda m: (m, 0)),
               pl.BlockSpec((_RB, _HD), lambda m: (m, 0)),
               pl.BlockSpec((_RB, _HD), lambda m: (m, 0))],
    out_shape=[jax.ShapeDtypeStruct((_N, 16), _f32),
               jax.ShapeDtypeStruct((_N, _HD), _f32),
               jax.ShapeDtypeStruct((_N, _HD), _f32)],
)


def _scale1_body(tlo_ref, thi_ref, dinv_ref, ylo_ref, yhi_ref):
    dv = dinv_ref[...][:, 0:1]
    d2 = dv * dv
    ylo_ref[...] = d2 * tlo_ref[...]
    yhi_ref[...] = d2 * thi_ref[...]


_scale1_tc = pl.pallas_call(
    _scale1_body,
    grid=(_N // _RB,),
    in_specs=[pl.BlockSpec((_RB, _HD), lambda m: (m, 0)),
              pl.BlockSpec((_RB, _HD), lambda m: (m, 0)),
              pl.BlockSpec((_RB, 16), lambda m: (m, 0))],
    out_specs=[pl.BlockSpec((_RB, _HD), lambda m: (m, 0)),
               pl.BlockSpec((_RB, _HD), lambda m: (m, 0))],
    out_shape=[jax.ShapeDtypeStruct((_N, _HD), _f32),
               jax.ShapeDtypeStruct((_N, _HD), _f32)],
)


_DB = 1024


def _dense_body(pu, alou, ahiu, blou, bhiu, du, nu,
                pi, aloi, ahii, bloi, bhii, di, ni,
                pj, aloj, ahij, bloj, bhij, dj, nj,
                sgn_ref, W0_ref, b0_ref, W1_ref, b1_ref,
                aW_ref, ab_ref, qv_ref, out_ref):
    W0 = W0_ref[...]
    b0 = b0_ref[...]
    W1 = W1_ref[...]
    b1 = b1_ref[...]
    aW = aW_ref[...]
    ab = ab_ref[...]
    qv = qv_ref[...]

    def seg(p_ref, alo_ref, ahi_ref, blo_ref, bhi_ref, d_ref, n_ref):
        d = d_ref[...][:, 0:1]
        t1 = jnp.concatenate([alo_ref[...], ahi_ref[...]], axis=1)
        t2 = jnp.concatenate([blo_ref[...], bhi_ref[...]], axis=1)
        zp = (p_ref[...] + d * t1 + d * t2) * (1.0 / 3.0)
        h = jnp.maximum(
            jnp.dot(n_ref[...], W0, preferred_element_type=_f32) + b0, 0.0)
        zn = jnp.maximum(
            jnp.dot(h, W1, preferred_element_type=_f32) + b1, 0.0)
        wp = jnp.sum(jnp.tanh(
            jnp.dot(zp, aW, preferred_element_type=_f32) + ab) * qv,
            axis=1, keepdims=True)
        wn = jnp.sum(jnp.tanh(
            jnp.dot(zn, aW, preferred_element_type=_f32) + ab) * qv,
            axis=1, keepdims=True)
        m = jnp.maximum(wp, wn)
        ep = jnp.exp(wp - m)
        en = jnp.exp(wn - m)
        al = ep / (ep + en)
        return al * zp + (1.0 - al) * zn

    Zu = seg(pu, alou, ahiu, blou, bhiu, du, nu)
    Zi = seg(pi, aloi, ahii, bloi, bhii, di, ni)
    Zj = seg(pj, aloj, ahij, bloj, bhij, dj, nj)
    ps = jnp.sum(Zu * Zi, axis=1, keepdims=True)
    ns = jnp.sum(Zu * Zj, axis=1, keepdims=True)
    x = sgn_ref[...] * ps - ns
    sig = 1.0 / (1.0 + jnp.exp(-x))
    part_sbpr = jnp.sum(jnp.log(_GAMMA + sig), axis=(0, 1), keepdims=True)
    part_reg = jnp.sum(Zu * Zu + Zi * Zi + Zj * Zj, axis=(0, 1), keepdims=True)
    part = (-part_sbpr + _REG * part_reg) / _B

    @pl.when(pl.program_id(0) == 0)
    def _():
        out_ref[...] = jnp.zeros((1, 1), _f32)

    out_ref[...] += part


def _seg_spec(cols, a):
    return pl.BlockSpec((_DB, cols), lambda m, a=a: (m + a * (_B // _DB), 0))


_w_spec = pl.BlockSpec((_DIM, _DIM), lambda m: (0, 0))
_b_spec = pl.BlockSpec((1, _DIM), lambda m: (0, 0))

_dense_tc = pl.pallas_call(
    _dense_body,
    grid=(_B // _DB,),
    in_specs=[sp
              for a in range(3)
              for sp in (_seg_spec(_DIM, a), _seg_spec(_HD, a),
                         _seg_spec(_HD, a), _seg_spec(_HD, a),
                         _seg_spec(_HD, a), _seg_spec(16, a),
                         _seg_spec(_DIM, a))]
    + [pl.BlockSpec((_DB, 1), lambda m: (m, 0)),
       _w_spec, _b_spec, _w_spec, _b_spec, _w_spec, _b_spec, _b_spec],
    out_specs=pl.BlockSpec((1, 1), lambda m: (0, 0)),
    out_shape=jax.ShapeDtypeStruct((1, 1), _f32),
)


def _pack_edges(a, pad):
    a = a.reshape(_NC, _NS, _EPW)
    a = jnp.pad(a, ((0, 0), (0, 0), (0, _EPAD - _EPW)), constant_values=pad)
    return a.reshape(_NC, _NS, _NCHUNK, _CH)


def kernel(emb_pos, emb_neg, W0, b0, W1, b1, attn_W, attn_b, q_W,
           sgn, u, i, j, edge_index):
    row, col = edge_index[0], edge_index[1]
    row_loc = lax.rem(row, _NUM_U)
    col_loc = lax.rem(col, _NUM_U)
    colg_pk = _pack_edges(col, 0)
    rowl_pk = _pack_edges(row_loc, _TRASH)
    coll_pk = _pack_edges(col_loc, _TRASH)

    zeros32 = jnp.zeros((_ZR, _HD), _f32)
    zeros16 = jnp.zeros((_ZR, 16), _f32)
    ones16 = jnp.ones((_CH, 16), _f32)

    deg16 = _deg_sc(coll_pk, zeros16, ones16)
    dinv16, y0lo, y0hi = _scale0_tc(deg16, emb_pos)
    t1lo, t1hi = _spmm_sc(y0lo, y0hi, colg_pk, rowl_pk, zeros32)
    y1lo, y1hi = _scale1_tc(t1lo, t1hi, dinv16)
    t2lo, t2hi = _spmm_sc(y1lo, y1hi, colg_pk, rowl_pk, zeros32)

    idx_pk = jnp.concatenate([u, _NUM_U + i, _NUM_U + j]).reshape(
        _NC * _NS, _GCH, _CH)
    posg, negg, t1log, t1hig, t2log, t2hig, dvg = _gather_sc(
        idx_pk, emb_pos, emb_neg, t1lo, t1hi, t2lo, t2hi, dinv16)

    sgn2 = sgn.reshape(_B, 1)
    segs = [posg, t1log, t1hig, t2log, t2hig, dvg, negg] * 3
    out = _dense_tc(
        *segs,
        sgn2, W0, b0.reshape(1, _DIM), W1, b1.reshape(1, _DIM),
        attn_W, attn_b.reshape(1, _DIM), q_W.reshape(1, _DIM))
    return out[0, 0]


# speedup vs baseline: 24.5133x; 1.0114x over previous
import functools

import jax
import jax.numpy as jnp
from jax import lax
from jax.experimental import pallas as pl
from jax.experimental.pallas import tpu as pltpu
from jax.experimental.pallas import tpu_sc as plsc

_NUM_U = 25000
_N = 50000
_DIM = 64
_E0 = 400000
_E = 800000
_B = 4096
_REG = 0.05
_GAMMA = 1e-10

_NC, _NS = 2, 16
_EPW = _E // (_NC * _NS)
_CH = 128
_NCHUNK = (_EPW + _CH - 1) // _CH
_EPAD = _NCHUNK * _CH
_HROWS = _NUM_U
_TRASH = _HROWS
_ACC_ROWS = _HROWS + 8
_ZR = 1568
_ZR_LAST = _HROWS - 15 * _ZR
_HD = _DIM // 2

_mesh = plsc.VectorSubcoreMesh(core_axis_name="c", subcore_axis_name="s")
_f32 = jnp.float32


@functools.partial(
    pl.kernel,
    compiler_params=pltpu.CompilerParams(use_tc_tiling_on_sc=False),
    out_type=jax.ShapeDtypeStruct((_N, 16), _f32),
    mesh=_mesh,
    scratch_types=[
        pltpu.VMEM((_NCHUNK, _CH), jnp.int32),
        pltpu.VMEM((_CH, 16), _f32),
        pltpu.VMEM_SHARED((_ACC_ROWS, 16), _f32),
        pltpu.SemaphoreType.DMA,
    ],
)
def _deg_sc(coll_pk, zeros16, ones16, deg_out, colv, onesv, dacc, sem):
    c = lax.axis_index("c")
    s = lax.axis_index("s")

    @pl.when(s < 15)
    def _():
        pltpu.sync_copy(zeros16, dacc.at[pl.ds(s * _ZR, _ZR)])

    @pl.when(s == 15)
    def _():
        pltpu.sync_copy(zeros16.at[pl.ds(0, _ZR_LAST)],
                        dacc.at[pl.ds(15 * _ZR, _ZR_LAST)])

    pltpu.sync_copy(coll_pk.at[c, s], colv)
    pltpu.sync_copy(ones16, onesv)
    plsc.subcore_barrier()

    @pl.loop(0, _NCHUNK, step=7)
    def _(k):
        for b in range(7):
            pltpu.async_copy(onesv, dacc.at[colv.at[k + b]], sem, add=True)
        for b in range(7):
            pltpu.make_async_copy(onesv, dacc.at[colv.at[k + b]], sem).wait()

    plsc.subcore_barrier()
    off = (1 - c) * _HROWS

    @pl.when(s < 15)
    def _():
        pltpu.sync_copy(dacc.at[pl.ds(s * _ZR, _ZR)],
                        deg_out.at[pl.ds(off + s * _ZR, _ZR)])

    @pl.when(s == 15)
    def _():
        pltpu.sync_copy(dacc.at[pl.ds(15 * _ZR, _ZR_LAST)],
                        deg_out.at[pl.ds(off + 15 * _ZR, _ZR_LAST)])


@functools.partial(
    pl.kernel,
    compiler_params=pltpu.CompilerParams(use_tc_tiling_on_sc=False),
    out_type=[jax.ShapeDtypeStruct((_N, _HD), _f32),
              jax.ShapeDtypeStruct((_N, _HD), _f32)],
    mesh=_mesh,
    scratch_types=[
        pltpu.VMEM((_NCHUNK, _CH), jnp.int32),
        pltpu.VMEM((_NCHUNK, _CH), jnp.int32),
        [pltpu.VMEM((_CH, _HD), _f32)] * 4,
        pltpu.VMEM_SHARED((_ACC_ROWS, _HD), _f32),
        [pltpu.SemaphoreType.DMA] * 4,
        [pltpu.SemaphoreType.DMA] * 4,
    ],
)
def _spmm_sc(ylo, yhi, colg_pk, rowl_pk, zeros32, tlo, thi,
             colv, rowv, vbs, acc, gsems, ssems):
    c = lax.axis_index("c")
    s = lax.axis_index("s")

    pltpu.sync_copy(colg_pk.at[c, s], colv)
    pltpu.sync_copy(rowl_pk.at[c, s], rowv)
    off = c * _HROWS

    for ytab, tout in ((ylo, tlo), (yhi, thi)):
        @pl.when(s < 15)
        def _():
            pltpu.sync_copy(zeros32, acc.at[pl.ds(s * _ZR, _ZR)])

        @pl.when(s == 15)
        def _():
            pltpu.sync_copy(zeros32.at[pl.ds(0, _ZR_LAST)],
                            acc.at[pl.ds(15 * _ZR, _ZR_LAST)])

        plsc.subcore_barrier()

        for b in range(4):
            pltpu.async_copy(ytab.at[colv.at[b]], vbs[b], gsems[b])

        @pl.loop(0, _NCHUNK, step=4)
        def _(k):
            for b in range(4):
                pltpu.make_async_copy(
                    ytab.at[colv.at[k + b]], vbs[b], gsems[b]).wait()
                pltpu.async_copy(
                    vbs[b], acc.at[rowv.at[k + b]], ssems[b], add=True)
            for b in range(4):
                pltpu.make_async_copy(
                    vbs[b], acc.at[rowv.at[k + b]], ssems[b]).wait()

                @pl.when(k + b + 4 < _NCHUNK)
                def _(b=b):
                    pltpu.async_copy(
                        ytab.at[colv.at[k + b + 4]], vbs[b], gsems[b])

        plsc.subcore_barrier()

        @pl.when(s < 15)
        def _():
            pltpu.sync_copy(acc.at[pl.ds(s * _ZR, _ZR)],
                            tout.at[pl.ds(off + s * _ZR, _ZR)])

        @pl.when(s == 15)
        def _():
            pltpu.sync_copy(acc.at[pl.ds(15 * _ZR, _ZR_LAST)],
                            tout.at[pl.ds(off + 15 * _ZR, _ZR_LAST)])

        plsc.subcore_barrier()


_G = 3 * _B
_GPW = _G // (_NC * _NS)
_GCH = _GPW // _CH


@functools.partial(
    pl.kernel,
    compiler_params=pltpu.CompilerParams(use_tc_tiling_on_sc=False),
    out_type=[
        jax.ShapeDtypeStruct((_G, _DIM), _f32),
        jax.ShapeDtypeStruct((_G, _DIM), _f32),
        jax.ShapeDtypeStruct((_G, _HD), _f32),
        jax.ShapeDtypeStruct((_G, _HD), _f32),
        jax.ShapeDtypeStruct((_G, _HD), _f32),
        jax.ShapeDtypeStruct((_G, _HD), _f32),
        jax.ShapeDtypeStruct((_G, 16), _f32),
    ],
    mesh=_mesh,
    scratch_types=[
        pltpu.VMEM((_GCH, _CH), jnp.int32),
        pltpu.VMEM((_CH, _DIM), _f32),
        pltpu.VMEM((_CH, _DIM), _f32),
        pltpu.VMEM((_CH, _HD), _f32),
        pltpu.VMEM((_CH, _HD), _f32),
        pltpu.VMEM((_CH, _HD), _f32),
        pltpu.VMEM((_CH, _HD), _f32),
        pltpu.VMEM((_CH, 16), _f32),
        pltpu.SemaphoreType.DMA,
    ],
)
def _gather_sc(idx_pk, pos_t, neg_t, t1lo_t, t1hi_t, t2lo_t, t2hi_t, dv_t,
               o_pos, o_neg, o_t1lo, o_t1hi, o_t2lo, o_t2hi, o_dv,
               idxv, bpos, bneg, b1lo, b1hi, b2lo, b2hi, bdv, sem):
    c = lax.axis_index("c")
    s = lax.axis_index("s")
    w = c * _NS + s
    pltpu.sync_copy(idx_pk.at[w], idxv)
    tabs = (pos_t, neg_t, t1lo_t, t1hi_t, t2lo_t, t2hi_t, dv_t)
    bufs = (bpos, bneg, b1lo, b1hi, b2lo, b2hi, bdv)
    outs = (o_pos, o_neg, o_t1lo, o_t1hi, o_t2lo, o_t2hi, o_dv)

    @pl.loop(0, _GCH)
    def _(k):
        for t, bf in zip(tabs, bufs):
            pltpu.async_copy(t.at[idxv.at[k]], bf, sem)
        for t, bf in zip(tabs, bufs):
            pltpu.make_async_copy(t.at[idxv.at[k]], bf, sem).wait()
        off = w * _GPW + k * _CH
        for bf, o in zip(bufs, outs):
            pltpu.sync_copy(bf, o.at[pl.ds(off, _CH)])


_RB = 10000


def _scale0_body(deg_ref, x_ref, dinv_ref, ylo_ref, yhi_ref):
    deg = deg_ref[...]
    dinv = jnp.where(deg > 0.0, lax.rsqrt(deg), 0.0)
    dinv_ref[...] = dinv
    d = dinv[:, 0:1]
    x = x_ref[...]
    ylo_ref[...] = d * x[:, :_HD]
    yhi_ref[...] = d * x[:, _HD:]


_scale0_tc = pl.pallas_call(
    _scale0_body,
    grid=(_N // _RB,),
    in_specs=[pl.BlockSpec((_RB, 16), lambda m: (m, 0)),
              pl.BlockSpec((_RB, _DIM), lambda m: (m, 0))],
    out_specs=[pl.BlockSpec((_RB, 16), lambda m: (m, 0)),
               pl.BlockSpec((_RB, _HD), lambda m: (m, 0)),
               pl.BlockSpec((_RB, _HD), lambda m: (m, 0))],
    out_shape=[jax.ShapeDtypeStruct((_N, 16), _f32),
               jax.ShapeDtypeStruct((_N, _HD), _f32),
               jax.ShapeDtypeStruct((_N, _HD), _f32)],
)


def _scale1_body(tlo_ref, thi_ref, dinv_ref, ylo_ref, yhi_ref):
    dv = dinv_ref[...][:, 0:1]
    d2 = dv * dv
    ylo_ref[...] = d2 * tlo_ref[...]
    yhi_ref[...] = d2 * thi_ref[...]


_scale1_tc = pl.pallas_call(
    _scale1_body,
    grid=(_N // _RB,),
    in_specs=[pl.BlockSpec((_RB, _HD), lambda m: (m, 0)),
              pl.BlockSpec((_RB, _HD), lambda m: (m, 0)),
              pl.BlockSpec((_RB, 16), lambda m: (m, 0))],
    out_specs=[pl.BlockSpec((_RB, _HD), lambda m: (m, 0)),
               pl.BlockSpec((_RB, _HD), lambda m: (m, 0))],
    out_shape=[jax.ShapeDtypeStruct((_N, _HD), _f32),
               jax.ShapeDtypeStruct((_N, _HD), _f32)],
)


_DB = 1024


def _dense_body(pu, alou, ahiu, blou, bhiu, du, nu,
                pi, aloi, ahii, bloi, bhii, di, ni,
                pj, aloj, ahij, bloj, bhij, dj, nj,
                sgn_ref, W0_ref, b0_ref, W1_ref, b1_ref,
                aW_ref, ab_ref, qv_ref, out_ref):
    W0 = W0_ref[...]
    b0 = b0_ref[...]
    W1 = W1_ref[...]
    b1 = b1_ref[...]
    aW = aW_ref[...]
    ab = ab_ref[...]
    qv = qv_ref[...]

    def seg(p_ref, alo_ref, ahi_ref, blo_ref, bhi_ref, d_ref, n_ref):
        d = d_ref[...][:, 0:1]
        t1 = jnp.concatenate([alo_ref[...], ahi_ref[...]], axis=1)
        t2 = jnp.concatenate([blo_ref[...], bhi_ref[...]], axis=1)
        zp = (p_ref[...] + d * t1 + d * t2) * (1.0 / 3.0)
        h = jnp.maximum(
            jnp.dot(n_ref[...], W0, preferred_element_type=_f32) + b0, 0.0)
        zn = jnp.maximum(
            jnp.dot(h, W1, preferred_element_type=_f32) + b1, 0.0)
        wp = jnp.sum(jnp.tanh(
            jnp.dot(zp, aW, preferred_element_type=_f32) + ab) * qv,
            axis=1, keepdims=True)
        wn = jnp.sum(jnp.tanh(
            jnp.dot(zn, aW, preferred_element_type=_f32) + ab) * qv,
            axis=1, keepdims=True)
        m = jnp.maximum(wp, wn)
        ep = jnp.exp(wp - m)
        en = jnp.exp(wn - m)
        al = ep / (ep + en)
        return al * zp + (1.0 - al) * zn

    Zu = seg(pu, alou, ahiu, blou, bhiu, du, nu)
    Zi = seg(pi, aloi, ahii, bloi, bhii, di, ni)
    Zj = seg(pj, aloj, ahij, bloj, bhij, dj, nj)
    ps = jnp.sum(Zu * Zi, axis=1, keepdims=True)
    ns = jnp.sum(Zu * Zj, axis=1, keepdims=True)
    x = sgn_ref[...] * ps - ns
    sig = 1.0 / (1.0 + jnp.exp(-x))
    part_sbpr = jnp.sum(jnp.log(_GAMMA + sig), axis=(0, 1), keepdims=True)
    part_reg = jnp.sum(Zu * Zu + Zi * Zi + Zj * Zj, axis=(0, 1), keepdims=True)
    part = (-part_sbpr + _REG * part_reg) / _B

    @pl.when(pl.program_id(0) == 0)
    def _():
        out_ref[...] = jnp.zeros((1, 1), _f32)

    out_ref[...] += part


def _seg_spec(cols, a):
    return pl.BlockSpec((_DB, cols), lambda m, a=a: (m + a * (_B // _DB), 0))


_w_spec = pl.BlockSpec((_DIM, _DIM), lambda m: (0, 0))
_b_spec = pl.BlockSpec((1, _DIM), lambda m: (0, 0))

_dense_tc = pl.pallas_call(
    _dense_body,
    grid=(_B // _DB,),
    in_specs=[sp
              for a in range(3)
              for sp in (_seg_spec(_DIM, a), _seg_spec(_HD, a),
                         _seg_spec(_HD, a), _seg_spec(_HD, a),
                         _seg_spec(_HD, a), _seg_spec(16, a),
                         _seg_spec(_DIM, a))]
    + [pl.BlockSpec((_DB, 1), lambda m: (m, 0)),
       _w_spec, _b_spec, _w_spec, _b_spec, _w_spec, _b_spec, _b_spec],
    out_specs=pl.BlockSpec((1, 1), lambda m: (0, 0)),
    out_shape=jax.ShapeDtypeStruct((1, 1), _f32),
)


def _pack_edges(a, pad):
    a = a.reshape(_NC, _NS, _EPW)
    a = jnp.pad(a, ((0, 0), (0, 0), (0, _EPAD - _EPW)), constant_values=pad)
    return a.reshape(_NC, _NS, _NCHUNK, _CH)


def kernel(emb_pos, emb_neg, W0, b0, W1, b1, attn_W, attn_b, q_W,
           sgn, u, i, j, edge_index):
    row, col = edge_index[0], edge_index[1]
    row_loc = lax.rem(row, _NUM_U)
    col_loc = lax.rem(col, _NUM_U)
    colg_pk = _pack_edges(col, 0)
    rowl_pk = _pack_edges(row_loc, _TRASH)
    coll_pk = _pack_edges(col_loc, _TRASH)

    zeros32 = jnp.zeros((_ZR, _HD), _f32)
    zeros16 = jnp.zeros((_ZR, 16), _f32)
    ones16 = jnp.ones((_CH, 16), _f32)

    deg16 = _deg_sc(coll_pk, zeros16, ones16)
    dinv16, y0lo, y0hi = _scale0_tc(deg16, emb_pos)
    t1lo, t1hi = _spmm_sc(y0lo, y0hi, colg_pk, rowl_pk, zeros32)
    y1lo, y1hi = _scale1_tc(t1lo, t1hi, dinv16)
    t2lo, t2hi = _spmm_sc(y1lo, y1hi, colg_pk, rowl_pk, zeros32)

    idx_pk = jnp.concatenate([u, _NUM_U + i, _NUM_U + j]).reshape(
        _NC * _NS, _GCH, _CH)
    posg, negg, t1log, t1hig, t2log, t2hig, dvg = _gather_sc(
        idx_pk, emb_pos, emb_neg, t1lo, t1hi, t2lo, t2hi, dinv16)

    sgn2 = sgn.reshape(_B, 1)
    segs = [posg, t1log, t1hig, t2log, t2hig, dvg, negg] * 3
    out = _dense_tc(
        *segs,
        sgn2, W0, b0.reshape(1, _DIM), W1, b1.reshape(1, _DIM),
        attn_W, attn_b.reshape(1, _DIM), q_W.reshape(1, _DIM))
    return out[0, 0]


# speedup vs baseline: 24.7888x; 1.0112x over previous
import functools

import jax
import jax.numpy as jnp
from jax import lax
from jax.experimental import pallas as pl
from jax.experimental.pallas import tpu as pltpu
from jax.experimental.pallas import tpu_sc as plsc

_NUM_U = 25000
_N = 50000
_DIM = 64
_E0 = 400000
_E = 800000
_B = 4096
_REG = 0.05
_GAMMA = 1e-10

_NC, _NS = 2, 16
_EPW = _E // (_NC * _NS)
_CH = 128
_NCHUNK = (_EPW + _CH - 1) // _CH
_EPAD = _NCHUNK * _CH
_HROWS = _NUM_U
_TRASH = _HROWS
_ACC_ROWS = _HROWS + 8
_ZR = 1568
_ZR_LAST = _HROWS - 15 * _ZR
_HD = _DIM // 2

_mesh = plsc.VectorSubcoreMesh(core_axis_name="c", subcore_axis_name="s")
_f32 = jnp.float32


@functools.partial(
    pl.kernel,
    compiler_params=pltpu.CompilerParams(use_tc_tiling_on_sc=False),
    out_type=jax.ShapeDtypeStruct((_N, 16), _f32),
    mesh=_mesh,
    scratch_types=[
        pltpu.VMEM((_NCHUNK, _CH), jnp.int32),
        pltpu.VMEM((_CH, 16), _f32),
        pltpu.VMEM_SHARED((_ACC_ROWS, 16), _f32),
        pltpu.SemaphoreType.DMA,
    ],
)
def _deg_sc(coll_pk, zeros16, ones16, deg_out, colv, onesv, dacc, sem):
    c = lax.axis_index("c")
    s = lax.axis_index("s")

    @pl.when(s < 15)
    def _():
        pltpu.sync_copy(zeros16, dacc.at[pl.ds(s * _ZR, _ZR)])

    @pl.when(s == 15)
    def _():
        pltpu.sync_copy(zeros16.at[pl.ds(0, _ZR_LAST)],
                        dacc.at[pl.ds(15 * _ZR, _ZR_LAST)])

    pltpu.sync_copy(coll_pk.at[c, s], colv)
    pltpu.sync_copy(ones16, onesv)
    plsc.subcore_barrier()

    @pl.loop(0, _NCHUNK, step=7)
    def _(k):
        for b in range(7):
            pltpu.async_copy(onesv, dacc.at[colv.at[k + b]], sem, add=True)
        for b in range(7):
            pltpu.make_async_copy(onesv, dacc.at[colv.at[k + b]], sem).wait()

    plsc.subcore_barrier()
    off = (1 - c) * _HROWS

    @pl.when(s < 15)
    def _():
        pltpu.sync_copy(dacc.at[pl.ds(s * _ZR, _ZR)],
                        deg_out.at[pl.ds(off + s * _ZR, _ZR)])

    @pl.when(s == 15)
    def _():
        pltpu.sync_copy(dacc.at[pl.ds(15 * _ZR, _ZR_LAST)],
                        deg_out.at[pl.ds(off + 15 * _ZR, _ZR_LAST)])


@functools.partial(
    pl.kernel,
    compiler_params=pltpu.CompilerParams(use_tc_tiling_on_sc=False),
    out_type=[jax.ShapeDtypeStruct((_N, _HD), _f32),
              jax.ShapeDtypeStruct((_N, _HD), _f32)],
    mesh=_mesh,
    scratch_types=[
        pltpu.VMEM((_NCHUNK, _CH), jnp.int32),
        pltpu.VMEM((_NCHUNK, _CH), jnp.int32),
        [pltpu.VMEM((_CH, _HD), _f32)] * 4,
        pltpu.VMEM_SHARED((_ACC_ROWS, _HD), _f32),
        [pltpu.SemaphoreType.DMA] * 4,
        [pltpu.SemaphoreType.DMA] * 4,
    ],
)
def _spmm_sc(ylo, yhi, colg_pk, rowl_pk, zeros32, tlo, thi,
             colv, rowv, vbs, acc, gsems, ssems):
    c = lax.axis_index("c")
    s = lax.axis_index("s")

    pltpu.sync_copy(colg_pk.at[c, s], colv)
    pltpu.sync_copy(rowl_pk.at[c, s], rowv)
    off = c * _HROWS

    for ytab, tout in ((ylo, tlo), (yhi, thi)):
        @pl.when(s < 15)
        def _():
            pltpu.sync_copy(zeros32, acc.at[pl.ds(s * _ZR, _ZR)])

        @pl.when(s == 15)
        def _():
            pltpu.sync_copy(zeros32.at[pl.ds(0, _ZR_LAST)],
                            acc.at[pl.ds(15 * _ZR, _ZR_LAST)])

        plsc.subcore_barrier()

        for b in range(4):
            pltpu.async_copy(ytab.at[colv.at[b]], vbs[b], gsems[b])

        @pl.loop(0, _NCHUNK, step=4)
        def _(k):
            for b in range(4):
                pltpu.make_async_copy(
                    ytab.at[colv.at[k + b]], vbs[b], gsems[b]).wait()
                pltpu.async_copy(
                    vbs[b], acc.at[rowv.at[k + b]], ssems[b], add=True)
            for b in range(4):
                pltpu.make_async_copy(
                    vbs[b], acc.at[rowv.at[k + b]], ssems[b]).wait()

                @pl.when(k + b + 4 < _NCHUNK)
                def _(b=b):
                    pltpu.async_copy(
                        ytab.at[colv.at[k + b + 4]], vbs[b], gsems[b])

        plsc.subcore_barrier()

        @pl.when(s < 15)
        def _():
            pltpu.sync_copy(acc.at[pl.ds(s * _ZR, _ZR)],
                            tout.at[pl.ds(off + s * _ZR, _ZR)])

        @pl.when(s == 15)
        def _():
            pltpu.sync_copy(acc.at[pl.ds(15 * _ZR, _ZR_LAST)],
                            tout.at[pl.ds(off + 15 * _ZR, _ZR_LAST)])

        plsc.subcore_barrier()


_G = 3 * _B
_GPW = _G // (_NC * _NS)
_GCH = _GPW // _CH


@functools.partial(
    pl.kernel,
    compiler_params=pltpu.CompilerParams(use_tc_tiling_on_sc=False),
    out_type=[
        jax.ShapeDtypeStruct((_G, _DIM), _f32),
        jax.ShapeDtypeStruct((_G, _DIM), _f32),
        jax.ShapeDtypeStruct((_G, _HD), _f32),
        jax.ShapeDtypeStruct((_G, _HD), _f32),
        jax.ShapeDtypeStruct((_G, _HD), _f32),
        jax.ShapeDtypeStruct((_G, _HD), _f32),
        jax.ShapeDtypeStruct((_G, 16), _f32),
    ],
    mesh=_mesh,
    scratch_types=[
        pltpu.VMEM((_GCH, _CH), jnp.int32),
        pltpu.VMEM((_CH, _DIM), _f32),
        pltpu.VMEM((_CH, _DIM), _f32),
        pltpu.VMEM((_CH, _HD), _f32),
        pltpu.VMEM((_CH, _HD), _f32),
        pltpu.VMEM((_CH, _HD), _f32),
        pltpu.VMEM((_CH, _HD), _f32),
        pltpu.VMEM((_CH, 16), _f32),
        pltpu.SemaphoreType.DMA,
    ],
)
def _gather_sc(idx_pk, pos_t, neg_t, t1lo_t, t1hi_t, t2lo_t, t2hi_t, dv_t,
               o_pos, o_neg, o_t1lo, o_t1hi, o_t2lo, o_t2hi, o_dv,
               idxv, bpos, bneg, b1lo, b1hi, b2lo, b2hi, bdv, sem):
    c = lax.axis_index("c")
    s = lax.axis_index("s")
    w = c * _NS + s
    pltpu.sync_copy(idx_pk.at[w], idxv)
    tabs = (pos_t, neg_t, t1lo_t, t1hi_t, t2lo_t, t2hi_t, dv_t)
    bufs = (bpos, bneg, b1lo, b1hi, b2lo, b2hi, bdv)
    outs = (o_pos, o_neg, o_t1lo, o_t1hi, o_t2lo, o_t2hi, o_dv)

    @pl.loop(0, _GCH)
    def _(k):
        for t, bf in zip(tabs, bufs):
            pltpu.async_copy(t.at[idxv.at[k]], bf, sem)
        for t, bf in zip(tabs, bufs):
            pltpu.make_async_copy(t.at[idxv.at[k]], bf, sem).wait()
        off = w * _GPW + k * _CH
        for bf, o in zip(bufs, outs):
            pltpu.sync_copy(bf, o.at[pl.ds(off, _CH)])


_RB = 10000


def _scale0_body(deg_ref, x_ref, ylo_ref, yhi_ref):
    deg = deg_ref[...]
    d = jnp.where(deg > 0.0, lax.rsqrt(deg), 0.0)[:, 0:1]
    x = x_ref[...]
    ylo_ref[...] = d * x[:, :_HD]
    yhi_ref[...] = d * x[:, _HD:]


_scale0_tc = pl.pallas_call(
    _scale0_body,
    grid=(_N // _RB,),
    in_specs=[pl.BlockSpec((_RB, 16), lambda m: (m, 0)),
              pl.BlockSpec((_RB, _DIM), lambda m: (m, 0))],
    out_specs=[pl.BlockSpec((_RB, _HD), lambda m: (m, 0)),
               pl.BlockSpec((_RB, _HD), lambda m: (m, 0))],
    out_shape=[jax.ShapeDtypeStruct((_N, _HD), _f32),
               jax.ShapeDtypeStruct((_N, _HD), _f32)],
)


def _scale1_body(tlo_ref, thi_ref, deg_ref, ylo_ref, yhi_ref):
    deg = deg_ref[...][:, 0:1]
    d2 = jnp.where(deg > 0.0, 1.0 / deg, 0.0)
    ylo_ref[...] = d2 * tlo_ref[...]
    yhi_ref[...] = d2 * thi_ref[...]


_scale1_tc = pl.pallas_call(
    _scale1_body,
    grid=(_N // _RB,),
    in_specs=[pl.BlockSpec((_RB, _HD), lambda m: (m, 0)),
              pl.BlockSpec((_RB, _HD), lambda m: (m, 0)),
              pl.BlockSpec((_RB, 16), lambda m: (m, 0))],
    out_specs=[pl.BlockSpec((_RB, _HD), lambda m: (m, 0)),
               pl.BlockSpec((_RB, _HD), lambda m: (m, 0))],
    out_shape=[jax.ShapeDtypeStruct((_N, _HD), _f32),
               jax.ShapeDtypeStruct((_N, _HD), _f32)],
)


_DB = 1024


def _dense_body(pu, alou, ahiu, blou, bhiu, du, nu,
                pi, aloi, ahii, bloi, bhii, di, ni,
                pj, aloj, ahij, bloj, bhij, dj, nj,
                sgn_ref, W0_ref, b0_ref, W1_ref, b1_ref,
                aW_ref, ab_ref, qv_ref, out_ref):
    W0 = W0_ref[...]
    b0 = b0_ref[...]
    W1 = W1_ref[...]
    b1 = b1_ref[...]
    aW = aW_ref[...]
    ab = ab_ref[...]
    qv = qv_ref[...]

    def seg(p_ref, alo_ref, ahi_ref, blo_ref, bhi_ref, d_ref, n_ref):
        deg = d_ref[...][:, 0:1]
        rsd = jnp.where(deg > 0.0, lax.rsqrt(deg), 0.0)
        t1 = jnp.concatenate([alo_ref[...], ahi_ref[...]], axis=1)
        t2 = jnp.concatenate([blo_ref[...], bhi_ref[...]], axis=1)
        zp = (p_ref[...] + rsd * (t1 + t2)) * (1.0 / 3.0)
        h = jnp.maximum(
            jnp.dot(n_ref[...], W0, preferred_element_type=_f32) + b0, 0.0)
        zn = jnp.maximum(
            jnp.dot(h, W1, preferred_element_type=_f32) + b1, 0.0)
        wp = jnp.sum(jnp.tanh(
            jnp.dot(zp, aW, preferred_element_type=_f32) + ab) * qv,
            axis=1, keepdims=True)
        wn = jnp.sum(jnp.tanh(
            jnp.dot(zn, aW, preferred_element_type=_f32) + ab) * qv,
            axis=1, keepdims=True)
        m = jnp.maximum(wp, wn)
        ep = jnp.exp(wp - m)
        en = jnp.exp(wn - m)
        al = ep / (ep + en)
        return al * zp + (1.0 - al) * zn

    Zu = seg(pu, alou, ahiu, blou, bhiu, du, nu)
    Zi = seg(pi, aloi, ahii, bloi, bhii, di, ni)
    Zj = seg(pj, aloj, ahij, bloj, bhij, dj, nj)
    ps = jnp.sum(Zu * Zi, axis=1, keepdims=True)
    ns = jnp.sum(Zu * Zj, axis=1, keepdims=True)
    x = sgn_ref[...] * ps - ns
    sig = 1.0 / (1.0 + jnp.exp(-x))
    part_sbpr = jnp.sum(jnp.log(_GAMMA + sig), axis=(0, 1), keepdims=True)
    part_reg = jnp.sum(Zu * Zu + Zi * Zi + Zj * Zj, axis=(0, 1), keepdims=True)
    part = (-part_sbpr + _REG * part_reg) / _B

    @pl.when(pl.program_id(0) == 0)
    def _():
        out_ref[...] = jnp.zeros((1, 1), _f32)

    out_ref[...] += part


def _seg_spec(cols, a):
    return pl.BlockSpec((_DB, cols), lambda m, a=a: (m + a * (_B // _DB), 0))


_w_spec = pl.BlockSpec((_DIM, _DIM), lambda m: (0, 0))
_b_spec = pl.BlockSpec((1, _DIM), lambda m: (0, 0))

_dense_tc = pl.pallas_call(
    _dense_body,
    grid=(_B // _DB,),
    in_specs=[sp
              for a in range(3)
              for sp in (_seg_spec(_DIM, a), _seg_spec(_HD, a),
                         _seg_spec(_HD, a), _seg_spec(_HD, a),
                         _seg_spec(_HD, a), _seg_spec(16, a),
                         _seg_spec(_DIM, a))]
    + [pl.BlockSpec((_DB, 1), lambda m: (m, 0)),
       _w_spec, _b_spec, _w_spec, _b_spec, _w_spec, _b_spec, _b_spec],
    out_specs=pl.BlockSpec((1, 1), lambda m: (0, 0)),
    out_shape=jax.ShapeDtypeStruct((1, 1), _f32),
)


def _pack_edges(a, pad):
    a = a.reshape(_NC, _NS, _EPW)
    a = jnp.pad(a, ((0, 0), (0, 0), (0, _EPAD - _EPW)), constant_values=pad)
    return a.reshape(_NC, _NS, _NCHUNK, _CH)


def kernel(emb_pos, emb_neg, W0, b0, W1, b1, attn_W, attn_b, q_W,
           sgn, u, i, j, edge_index):
    row, col = edge_index[0], edge_index[1]
    row_loc = lax.rem(row, _NUM_U)
    col_loc = lax.rem(col, _NUM_U)
    colg_pk = _pack_edges(col, 0)
    rowl_pk = _pack_edges(row_loc, _TRASH)
    coll_pk = _pack_edges(col_loc, _TRASH)

    zeros32 = jnp.zeros((_ZR, _HD), _f32)
    zeros16 = jnp.zeros((_ZR, 16), _f32)
    ones16 = jnp.ones((_CH, 16), _f32)

    deg16 = _deg_sc(coll_pk, zeros16, ones16)
    y0lo, y0hi = _scale0_tc(deg16, emb_pos)
    t1lo, t1hi = _spmm_sc(y0lo, y0hi, colg_pk, rowl_pk, zeros32)
    y1lo, y1hi = _scale1_tc(t1lo, t1hi, deg16)
    t2lo, t2hi = _spmm_sc(y1lo, y1hi, colg_pk, rowl_pk, zeros32)

    idx_pk = jnp.concatenate([u, _NUM_U + i, _NUM_U + j]).reshape(
        _NC * _NS, _GCH, _CH)
    posg, negg, t1log, t1hig, t2log, t2hig, dvg = _gather_sc(
        idx_pk, emb_pos, emb_neg, t1lo, t1hi, t2lo, t2hi, deg16)

    sgn2 = sgn.reshape(_B, 1)
    segs = [posg, t1log, t1hig, t2log, t2hig, dvg, negg] * 3
    out = _dense_tc(
        *segs,
        sgn2, W0, b0.reshape(1, _DIM), W1, b1.reshape(1, _DIM),
        attn_W, attn_b.reshape(1, _DIM), q_W.reshape(1, _DIM))
    return out[0, 0]
